# baseline (speedup 1.0000x reference)
import functools

import jax
import jax.numpy as jnp
from jax import lax
from jax.experimental import pallas as pl
from jax.experimental.pallas import tpu as pltpu
from jax.experimental.pallas import tpu_sc as plsc


def _ln_act(t, g, be, relu=True):
    mu = jnp.mean(t, axis=-1, keepdims=True)
    d = t - mu
    var = jnp.mean(d * d, axis=-1, keepdims=True)
    y = d / jnp.sqrt(var + 1e-5) * g + be
    return jnp.maximum(y, 0.0) if relu else y


def _rne_bf16_bits(x):
    b = jax.lax.bitcast_convert_type(x, jnp.int32)
    return ((b + 0x7FFF + ((b >> 16) & 1)) >> 16) & 0xFFFF


def _ea_mlp_body(ea_ref, w1, b1, g1, be1, w2, b2, g2, be2, out_ref):
    t = jnp.dot(ea_ref[...], w1[...], preferred_element_type=jnp.float32) + b1[...]
    t = _ln_act(t, g1[...], be1[...])
    t = jnp.dot(t, w2[...], preferred_element_type=jnp.float32) + b2[...]
    t = _ln_act(t, g2[...], be2[...])
    half = t.shape[1] // 2
    lo = _rne_bf16_bits(t[:, :half])
    hi = _rne_bf16_bits(t[:, half:])
    out_ref[...] = lo | (hi << 16)


def _ea_mlp(edge_attr, p):
    E, EA = edge_attr.shape
    HID = p['W_ea1'].shape[1]
    BLK = 2560
    grid = E // BLK
    row = lambda shape: pl.BlockSpec(shape, lambda i: (0, 0))
    return pl.pallas_call(
        _ea_mlp_body,
        grid=(grid,),
        in_specs=[
            pl.BlockSpec((BLK, EA), lambda i: (i, 0)),
            row((EA, HID)), row((1, HID)), row((1, HID)), row((1, HID)),
            row((HID, HID)), row((1, HID)), row((1, HID)), row((1, HID)),
        ],
        out_specs=pl.BlockSpec((BLK, HID // 2), lambda i: (i, 0)),
        out_shape=jax.ShapeDtypeStruct((E, HID // 2), jnp.int32),
    )(edge_attr, p['W_ea1'], p['b_ea1'].reshape(1, -1), p['g_ea1'].reshape(1, -1),
      p['be_ea1'].reshape(1, -1), p['W_ea2'], p['b_ea2'].reshape(1, -1),
      p['g_ea2'].reshape(1, -1), p['be_ea2'].reshape(1, -1))


def _pack_half(hv):
    half = hv.shape[1] // 2
    return _rne_bf16_bits(hv[:, :half]) | (_rne_bf16_bits(hv[:, half:]) << 16)


def _mlp_first_body(x_ref, w, b, g, be, out_ref, out16_ref):
    t = jnp.dot(x_ref[...], w[...], preferred_element_type=jnp.float32) + b[...]
    hv = _ln_act(t, g[...], be[...])
    out_ref[...] = hv
    out16_ref[...] = _pack_half(hv)


def _mlp_fused_body(x_ref, m0_ref, m1_ref, w, b, g, be, out_ref, out16_ref):
    s = x_ref[...] + m0_ref[...] + m1_ref[...]
    t = jnp.dot(s, w[...], preferred_element_type=jnp.float32) + b[...]
    hv = _ln_act(t, g[...], be[...])
    out_ref[...] = hv
    out16_ref[...] = _pack_half(hv)


def _node_mlp(x, msgs, w, b, g, be):
    N, HID = x.shape
    BLK = 2000
    grid = N // BLK
    row = lambda shape: pl.BlockSpec(shape, lambda i: (0, 0))
    blk = pl.BlockSpec((BLK, HID), lambda i: (i, 0))
    if msgs is None:
        body, ins = _mlp_first_body, [x]
        in_specs = [blk]
    else:
        body, ins = _mlp_fused_body, [x, msgs[0], msgs[1]]
        in_specs = [blk, blk, blk]
    in_specs += [row((HID, HID)), row((1, HID)), row((1, HID)), row((1, HID))]
    return pl.pallas_call(
        body,
        grid=(grid,),
        in_specs=in_specs,
        out_specs=[blk, pl.BlockSpec((BLK, HID // 2), lambda i: (i, 0))],
        out_shape=[jax.ShapeDtypeStruct((N, HID), jnp.float32),
                   jax.ShapeDtypeStruct((N, HID // 2), jnp.int32)],
    )(*ins, w, b.reshape(1, -1), g.reshape(1, -1), be.reshape(1, -1))


def _pool_out_body(x_ref, m0_ref, m1_ref, batch_ref, w, b, out_ref, acc_ref,
                   *, n_graphs, grid):
    i = pl.program_id(0)

    @pl.when(i == 0)
    def _():
        acc_ref[...] = jnp.zeros_like(acc_ref)

    s = x_ref[...] + m0_ref[...] + m1_ref[...]
    bvec = batch_ref[0, 0, :]
    gid = jax.lax.broadcasted_iota(jnp.int32, (bvec.shape[0], n_graphs), 1)
    onehot = (bvec[:, None] == gid).astype(jnp.float32)
    acc_ref[...] += jax.lax.dot_general(
        onehot, s, (((0,), (0,)), ((), ())), preferred_element_type=jnp.float32)

    @pl.when(i == grid - 1)
    def _():
        out_ref[...] = jnp.dot(acc_ref[...], w[...],
                               preferred_element_type=jnp.float32) + b[...]


def _pool_out(x, msgs, batch, w_out, b_out):
    N, HID = x.shape
    OUT = w_out.shape[1]
    n_graphs = 64
    BLK = 2000
    grid = N // BLK
    row = lambda shape: pl.BlockSpec(shape, lambda i: (0, 0))
    blk = pl.BlockSpec((BLK, HID), lambda i: (i, 0))
    batch3 = batch.reshape(grid, 1, BLK)
    body = functools.partial(_pool_out_body, n_graphs=n_graphs, grid=grid)
    return pl.pallas_call(
        body,
        grid=(grid,),
        in_specs=[
            blk, blk, blk,
            pl.BlockSpec((1, 1, BLK), lambda i: (i, 0, 0)),
            row((HID, OUT)), row((1, OUT)),
        ],
        out_specs=pl.BlockSpec((n_graphs, OUT), lambda i: (0, 0)),
        out_shape=jax.ShapeDtypeStruct((n_graphs, OUT), jnp.float32),
        scratch_shapes=[pltpu.VMEM((n_graphs, HID), jnp.float32)],
    )(x, msgs[0], msgs[1], batch3, w_out, b_out.reshape(1, -1))


_B = 40
_NS = 16
_NC = 2
_NR = 5
_NE = 2
_NI = 10


def _conv_sc(h, ea, src, dst):
    N = h.shape[0]
    HID = h.shape[1] * 2
    E = ea.shape[0]
    epw = E // (_NC * _NS)
    ch = epw // _B
    nrch = N // _B
    full = -(-nrch // _NS)
    nk = HID // 16
    ng = HID // 32

    def body(h_hbm, ea_hbm, src_hbm, dst_hbm, m0_hbm, m1_hbm,
             rows_v, ea_v, stg_v, di_v, si_v, acc, sem_g, sem_e, sem_id,
             sem_is):
        c = lax.axis_index("c")
        s = lax.axis_index("s")
        w = c * _NS + s
        ncp = jnp.where(s == _NS - 1, nrch - full * (_NS - 1), full)
        row0 = s * full * _B
        base = w * ch

        def start_idx(j, t):
            e0 = (base + j) * _B
            pltpu.async_copy(dst_hbm.at[pl.ds(e0, _B)], di_v[t], sem_id.at[t])
            pltpu.async_copy(src_hbm.at[pl.ds(e0, _B)], si_v[t], sem_is.at[t])

        def wait_idx(t):
            pltpu.make_async_copy(dst_hbm.at[pl.ds(0, _B)], di_v[t],
                                  sem_id.at[t]).wait()
            pltpu.make_async_copy(src_hbm.at[pl.ds(0, _B)], si_v[t],
                                  sem_is.at[t]).wait()

        def start_gather(t, b):
            pltpu.async_copy(h_hbm.at[di_v[t]], rows_v[b], sem_g.at[b])

        def start_ea(j, e):
            pltpu.async_copy(ea_hbm.at[pl.ds((base + j) * _B, _B)], ea_v[e],
                             sem_e.at[e])

        for t in range(_NI):
            start_idx(t, t)

        @pl.loop(0, _B)
        def _zrow(r):
            for k in range(nk):
                stg_v[r, pl.ds(k * 16, 16)] = jnp.zeros((16,), jnp.float32)

        @pl.loop(0, ncp)
        def _zcp(i):
            pltpu.sync_copy(stg_v, acc.at[pl.ds(row0 + i * _B, _B)])

        plsc.subcore_barrier()

        for b in range(_NR):
            wait_idx(b)
            start_gather(b, b)
        for e in range(_NE):
            start_ea(e, e)

        @pl.loop(0, ch // _NI)
        def _grp(g):
            for u in range(_NI):
                j = g * _NI + u
                b = u % _NR
                e = u % _NE
                pltpu.make_async_copy(h_hbm.at[di_v[u]], rows_v[b],
                                      sem_g.at[b]).wait()
                pltpu.make_async_copy(ea_hbm.at[pl.ds(0, _B)], ea_v[e],
                                      sem_e.at[e]).wait()

                @plsc.parallel_loop(0, _B, unroll=4)
                def _row(r):
                    for m in range(ng):
                        sl = pl.ds(m * 16, 16)
                        wr = rows_v[b][r, sl]
                        we = ea_v[e][r, sl]
                        rlo = lax.bitcast_convert_type(wr << 16, jnp.float32)
                        rhi = lax.bitcast_convert_type(
                            wr & jnp.int32(-65536), jnp.float32)
                        elo = lax.bitcast_convert_type(we << 16, jnp.float32)
                        ehi = lax.bitcast_convert_type(
                            we & jnp.int32(-65536), jnp.float32)
                        stg_v[r, sl] = rlo * elo
                        stg_v[r, pl.ds(HID // 2 + m * 16, 16)] = rhi * ehi

                pltpu.sync_copy(stg_v, acc.at[si_v[u]], add=True)

                @pl.when(j + _NI < ch)
                def _():
                    start_idx(j + _NI, u)

                @pl.when(j + _NR < ch)
                def _():
                    wait_idx((u + _NR) % _NI)
                    start_gather((u + _NR) % _NI, b)

                @pl.when(j + _NE < ch)
                def _():
                    start_ea(j + _NE, e)

        plsc.subcore_barrier()

        @pl.when(c == 0)
        def _():
            @pl.loop(0, ncp)
            def _wb0(i):
                r0 = row0 + i * _B
                pltpu.sync_copy(acc.at[pl.ds(r0, _B)], m0_hbm.at[pl.ds(r0, _B)])

        @pl.when(c == 1)
        def _():
            @pl.loop(0, ncp)
            def _wb1(i):
                r0 = row0 + i * _B
                pltpu.sync_copy(acc.at[pl.ds(r0, _B)], m1_hbm.at[pl.ds(r0, _B)])

    mesh = plsc.VectorSubcoreMesh(core_axis_name="c", subcore_axis_name="s")
    fn = pl.kernel(
        body,
        compiler_params=pltpu.CompilerParams(use_tc_tiling_on_sc=False),
        out_type=[jax.ShapeDtypeStruct((N, HID), jnp.float32),
                  jax.ShapeDtypeStruct((N, HID), jnp.float32)],
        mesh=mesh,
        scratch_types=[
            [pltpu.VMEM((_B, HID // 2), jnp.int32) for _ in range(_NR)],
            [pltpu.VMEM((_B, HID // 2), jnp.int32) for _ in range(_NE)],
            pltpu.VMEM((_B, HID), jnp.float32),
            [pltpu.VMEM((_B,), jnp.int32) for _ in range(_NI)],
            [pltpu.VMEM((_B,), jnp.int32) for _ in range(_NI)],
            pltpu.VMEM_SHARED((N, HID), jnp.float32),
            pltpu.SemaphoreType.DMA((_NR,)),
            pltpu.SemaphoreType.DMA((_NE,)),
            pltpu.SemaphoreType.DMA((_NI,)),
            pltpu.SemaphoreType.DMA((_NI,)),
        ],
    )
    return fn(h, ea, src, dst)


LAYER = 3


def kernel(x, edge_index, edge_attr, batch, params):
    p = params
    eap = _ea_mlp(edge_attr, p)
    src = edge_index[0]
    dst = edge_index[1]

    h, hp = _node_mlp(x, None, p['W_mlp'][0], p['b_mlp'][0], p['g_mlp'][0],
                      p['be_mlp'][0])
    msgs = None
    for i in range(LAYER):
        if i > 0:
            h, hp = _node_mlp(h, msgs, p['W_mlp'][i], p['b_mlp'][i],
                              p['g_mlp'][i], p['be_mlp'][i])
        msgs = _conv_sc(hp, eap, src, dst)

    return _pool_out(h, msgs, batch, p['W_out'], p['b_out'])


# speedup vs baseline: 1.0452x; 1.0452x over previous
import functools

import jax
import jax.numpy as jnp
from jax import lax
from jax.experimental import pallas as pl
from jax.experimental.pallas import tpu as pltpu
from jax.experimental.pallas import tpu_sc as plsc


def _ln_act(t, g, be):
    mu = jnp.mean(t, axis=-1, keepdims=True)
    d = t - mu
    var = jnp.mean(d * d, axis=-1, keepdims=True)
    y = d / jnp.sqrt(var + 1e-5) * g + be
    return jnp.maximum(y, 0.0)


def _ea_mlp_body(ea_ref, w1, b1, g1, be1, w2, b2, g2, be2, out_ref):
    t = jnp.dot(ea_ref[...], w1[...], preferred_element_type=jnp.float32) + b1[...]
    t = _ln_act(t, g1[...], be1[...])
    t = jnp.dot(t, w2[...], preferred_element_type=jnp.float32) + b2[...]
    out_ref[...] = _ln_act(t, g2[...], be2[...])


def _ea_mlp(edge_attr, p):
    E, EA = edge_attr.shape
    HID = p['W_ea1'].shape[1]
    BLK = 2560
    grid = E // BLK
    row = lambda shape: pl.BlockSpec(shape, lambda i: (0, 0))
    return pl.pallas_call(
        _ea_mlp_body,
        grid=(grid,),
        in_specs=[
            pl.BlockSpec((BLK, EA), lambda i: (i, 0)),
            row((EA, HID)), row((1, HID)), row((1, HID)), row((1, HID)),
            row((HID, HID)), row((1, HID)), row((1, HID)), row((1, HID)),
        ],
        out_specs=pl.BlockSpec((BLK, HID), lambda i: (i, 0)),
        out_shape=jax.ShapeDtypeStruct((E, HID), jnp.float32),
    )(edge_attr, p['W_ea1'], p['b_ea1'].reshape(1, -1), p['g_ea1'].reshape(1, -1),
      p['be_ea1'].reshape(1, -1), p['W_ea2'], p['b_ea2'].reshape(1, -1),
      p['g_ea2'].reshape(1, -1), p['be_ea2'].reshape(1, -1))


def _mlp_first_body(x_ref, w, b, g, be, out_ref):
    t = jnp.dot(x_ref[...], w[...], preferred_element_type=jnp.float32) + b[...]
    out_ref[...] = _ln_act(t, g[...], be[...])


def _mlp_fused_body(x_ref, m0_ref, m1_ref, w, b, g, be, out_ref):
    s = x_ref[...] + m0_ref[...] + m1_ref[...]
    t = jnp.dot(s, w[...], preferred_element_type=jnp.float32) + b[...]
    out_ref[...] = _ln_act(t, g[...], be[...])


def _node_mlp(x, msgs, w, b, g, be):
    N, HID = x.shape
    BLK = 2000
    grid = N // BLK
    row = lambda shape: pl.BlockSpec(shape, lambda i: (0, 0))
    blk = pl.BlockSpec((BLK, HID), lambda i: (i, 0))
    if msgs is None:
        body, ins = _mlp_first_body, [x]
        in_specs = [blk]
    else:
        body, ins = _mlp_fused_body, [x, msgs[0], msgs[1]]
        in_specs = [blk, blk, blk]
    in_specs += [row((HID, HID)), row((1, HID)), row((1, HID)), row((1, HID))]
    return pl.pallas_call(
        body,
        grid=(grid,),
        in_specs=in_specs,
        out_specs=blk,
        out_shape=jax.ShapeDtypeStruct((N, HID), jnp.float32),
    )(*ins, w, b.reshape(1, -1), g.reshape(1, -1), be.reshape(1, -1))


def _pool_out_body(x_ref, m0_ref, m1_ref, batch_ref, w, b, out_ref, acc_ref,
                   *, n_graphs, grid):
    i = pl.program_id(0)

    @pl.when(i == 0)
    def _():
        acc_ref[...] = jnp.zeros_like(acc_ref)

    s = x_ref[...] + m0_ref[...] + m1_ref[...]
    bvec = batch_ref[0, 0, :]
    gid = jax.lax.broadcasted_iota(jnp.int32, (bvec.shape[0], n_graphs), 1)
    onehot = (bvec[:, None] == gid).astype(jnp.float32)
    acc_ref[...] += jax.lax.dot_general(
        onehot, s, (((0,), (0,)), ((), ())), preferred_element_type=jnp.float32)

    @pl.when(i == grid - 1)
    def _():
        out_ref[...] = jnp.dot(acc_ref[...], w[...],
                               preferred_element_type=jnp.float32) + b[...]


def _pool_out(x, msgs, batch, w_out, b_out):
    N, HID = x.shape
    OUT = w_out.shape[1]
    n_graphs = 64
    BLK = 2000
    grid = N // BLK
    row = lambda shape: pl.BlockSpec(shape, lambda i: (0, 0))
    blk = pl.BlockSpec((BLK, HID), lambda i: (i, 0))
    batch3 = batch.reshape(grid, 1, BLK)
    body = functools.partial(_pool_out_body, n_graphs=n_graphs, grid=grid)
    return pl.pallas_call(
        body,
        grid=(grid,),
        in_specs=[
            blk, blk, blk,
            pl.BlockSpec((1, 1, BLK), lambda i: (i, 0, 0)),
            row((HID, OUT)), row((1, OUT)),
        ],
        out_specs=pl.BlockSpec((n_graphs, OUT), lambda i: (0, 0)),
        out_shape=jax.ShapeDtypeStruct((n_graphs, OUT), jnp.float32),
        scratch_shapes=[pltpu.VMEM((n_graphs, HID), jnp.float32)],
    )(x, msgs[0], msgs[1], batch3, w_out, b_out.reshape(1, -1))


_B = 40
_NS = 16
_NC = 2
_NR = 5
_NE = 2
_NI = 10


def _conv_sc(h, ea, src, dst):
    N, HID = h.shape
    E = ea.shape[0]
    epw = E // (_NC * _NS)
    ch = epw // _B
    nrch = N // _B
    full = -(-nrch // _NS)
    nk = HID // 16

    def body(h_hbm, ea_hbm, src_hbm, dst_hbm, m0_hbm, m1_hbm,
             rows_v, ea_v, di_v, si_v, acc, sem_g, sem_e, sem_id, sem_is):
        c = lax.axis_index("c")
        s = lax.axis_index("s")
        w = c * _NS + s
        ncp = jnp.where(s == _NS - 1, nrch - full * (_NS - 1), full)
        row0 = s * full * _B
        base = w * ch

        def start_idx(j, t):
            e0 = (base + j) * _B
            pltpu.async_copy(dst_hbm.at[pl.ds(e0, _B)], di_v[t], sem_id.at[t])
            pltpu.async_copy(src_hbm.at[pl.ds(e0, _B)], si_v[t], sem_is.at[t])

        def wait_idx(t):
            pltpu.make_async_copy(dst_hbm.at[pl.ds(0, _B)], di_v[t],
                                  sem_id.at[t]).wait()
            pltpu.make_async_copy(src_hbm.at[pl.ds(0, _B)], si_v[t],
                                  sem_is.at[t]).wait()

        def start_gather(t, b):
            pltpu.async_copy(h_hbm.at[di_v[t]], rows_v[b], sem_g.at[b])

        def start_ea(j, e):
            pltpu.async_copy(ea_hbm.at[pl.ds((base + j) * _B, _B)], ea_v[e],
                             sem_e.at[e])

        for t in range(_NI):
            start_idx(t, t)

        @pl.loop(0, _B)
        def _zrow(r):
            for k in range(nk):
                rows_v[0][r, pl.ds(k * 16, 16)] = jnp.zeros((16,), jnp.float32)

        @pl.loop(0, ncp)
        def _zcp(i):
            pltpu.sync_copy(rows_v[0], acc.at[pl.ds(row0 + i * _B, _B)])

        plsc.subcore_barrier()

        for b in range(_NR):
            wait_idx(b)
            start_gather(b, b)
        for e in range(_NE):
            start_ea(e, e)

        @pl.loop(0, ch // _NI)
        def _grp(g):
            for u in range(_NI):
                j = g * _NI + u
                b = u % _NR
                e = u % _NE
                pltpu.make_async_copy(h_hbm.at[di_v[u]], rows_v[b],
                                      sem_g.at[b]).wait()
                pltpu.make_async_copy(ea_hbm.at[pl.ds(0, _B)], ea_v[e],
                                      sem_e.at[e]).wait()

                @plsc.parallel_loop(0, _B, unroll=4)
                def _row(r):
                    for k in range(nk):
                        sl = pl.ds(k * 16, 16)
                        rows_v[b][r, sl] = rows_v[b][r, sl] * ea_v[e][r, sl]

                pltpu.sync_copy(rows_v[b], acc.at[si_v[u]], add=True)

                @pl.when(j + _NI < ch)
                def _():
                    start_idx(j + _NI, u)

                @pl.when(j + _NR < ch)
                def _():
                    wait_idx((u + _NR) % _NI)
                    start_gather((u + _NR) % _NI, b)

                @pl.when(j + _NE < ch)
                def _():
                    start_ea(j + _NE, e)

        plsc.subcore_barrier()

        @pl.when(c == 0)
        def _():
            @pl.loop(0, ncp)
            def _wb0(i):
                r0 = row0 + i * _B
                pltpu.sync_copy(acc.at[pl.ds(r0, _B)], m0_hbm.at[pl.ds(r0, _B)])

        @pl.when(c == 1)
        def _():
            @pl.loop(0, ncp)
            def _wb1(i):
                r0 = row0 + i * _B
                pltpu.sync_copy(acc.at[pl.ds(r0, _B)], m1_hbm.at[pl.ds(r0, _B)])

    mesh = plsc.VectorSubcoreMesh(core_axis_name="c", subcore_axis_name="s")
    fn = pl.kernel(
        body,
        out_type=[jax.ShapeDtypeStruct((N, HID), jnp.float32),
                  jax.ShapeDtypeStruct((N, HID), jnp.float32)],
        mesh=mesh,
        scratch_types=[
            [pltpu.VMEM((_B, HID), jnp.float32) for _ in range(_NR)],
            [pltpu.VMEM((_B, HID), jnp.float32) for _ in range(_NE)],
            [pltpu.VMEM((_B,), jnp.int32) for _ in range(_NI)],
            [pltpu.VMEM((_B,), jnp.int32) for _ in range(_NI)],
            pltpu.VMEM_SHARED((N, HID), jnp.float32),
            pltpu.SemaphoreType.DMA((_NR,)),
            pltpu.SemaphoreType.DMA((_NE,)),
            pltpu.SemaphoreType.DMA((_NI,)),
            pltpu.SemaphoreType.DMA((_NI,)),
        ],
    )
    return fn(h, ea, src, dst)


LAYER = 3


def kernel(x, edge_index, edge_attr, batch, params):
    p = params
    ea = _ea_mlp(edge_attr, p)
    src = edge_index[0]
    dst = edge_index[1]

    h = _node_mlp(x, None, p['W_mlp'][0], p['b_mlp'][0], p['g_mlp'][0],
                  p['be_mlp'][0])
    msgs = None
    for i in range(LAYER):
        if i > 0:
            h = _node_mlp(h, msgs, p['W_mlp'][i], p['b_mlp'][i],
                          p['g_mlp'][i], p['be_mlp'][i])
        msgs = _conv_sc(h, ea, src, dst)

    return _pool_out(h, msgs, batch, p['W_out'], p['b_out'])
